# baseline (speedup 1.0000x reference)
import functools

import jax
import jax.numpy as jnp
from jax import lax
from jax.experimental import pallas as pl
from jax.experimental.pallas import tpu as pltpu
from jax.experimental.pallas import tpu_sc as plsc

N = 1048576
K = 4096
NC = 2
NS = 16
L = 16
M = N // NS
CH = 4096
NCH = M // CH
NB1 = 1024
NB2 = 4096
U = 8

_mesh = plsc.VectorSubcoreMesh(core_axis_name="c", subcore_axis_name="s")
_cparams = pltpu.CompilerParams(needs_layout_passes=False)


def _zero_ref(ref, n_words):
    zero16 = jnp.zeros((L,), jnp.float32)
    blk = 32 * L

    def z(i, _):
        for u in range(32):
            ref[pl.ds(i * blk + u * L, L)] = zero16
        return 0

    lax.fori_loop(0, n_words // blk, z, 0)


def _tree_merge_rows(src, n_rows, row_stride, dst, nb):
    def merge(j, _):
        vs = [src[pl.ds(r * row_stride + j * L, L)] for r in range(n_rows)]
        while len(vs) > 1:
            vs = [vs[i] + vs[i + 1] for i in range(0, len(vs) - 1, 2)] + (
                [vs[-1]] if len(vs) % 2 else [])
        dst[pl.ds(j * L, L)] = vs[0]
        return 0

    lax.fori_loop(0, nb // L, merge, 0)


def _find_bin(mc, threshold, nb):
    def body(jj, carry):
        cnt_acc, sum_carry = carry
        j = nb // L - 1 - jj
        v = mc[pl.ds(j * L, L)]
        sfx = lax.rev(jnp.cumsum(lax.rev(v, (0,))), (0,)) + sum_carry
        ge = sfx >= threshold
        cnt_acc = cnt_acc + plsc.all_reduce_population_count(ge)
        return cnt_acc, sum_carry + jnp.sum(v)

    cnt, _ = lax.fori_loop(
        0, nb // L, body, (jnp.zeros((L,), jnp.int32), jnp.float32(0.0))
    )
    return cnt - 1


def _select_bin(mc, ms, threshold, nb):
    zero = jnp.zeros((L,), jnp.float32)

    def body(jj, carry):
        sum_carry, cgt, sgt, ceq, seq = carry
        j = nb // L - 1 - jj
        vc = mc[pl.ds(j * L, L)]
        vs = ms[pl.ds(j * L, L)]
        sfx = lax.rev(jnp.cumsum(lax.rev(vc, (0,))), (0,)) + sum_carry
        ge = sfx >= threshold
        lt = sfx < threshold
        eq = jnp.logical_and(ge, (sfx - vc) < threshold)
        return (sum_carry + jnp.sum(vc),
                cgt + jnp.where(lt, vc, zero), sgt + jnp.where(lt, vs, zero),
                ceq + jnp.where(eq, vc, zero), seq + jnp.where(eq, vs, zero))

    _, cgt, sgt, ceq, seq = lax.fori_loop(
        0, nb // L, body, (jnp.float32(0.0), zero, zero, zero, zero))
    return jnp.sum(cgt), jnp.sum(sgt), jnp.sum(ceq), jnp.sum(seq)


def _body(x_hbm, y_hbm, loss_hbm,
          xbuf0, ybuf0, xbuf1, ybuf1, bce, h1, m1c, m1s, h2c, h2s,
          g2c, g2s, shc, shs, sh2c, sh2s, obuf, semi0, semi1):
    cid = lax.axis_index("c")
    sid = lax.axis_index("s")
    base = sid * M
    lane = lax.iota(jnp.int32, L)
    ones = jnp.ones((L,), jnp.float32)
    bufs = [(xbuf0, ybuf0, semi0), (xbuf1, ybuf1, semi1)]
    descs_in = [None, None]

    def start_in(ch):
        p = ch & 1
        xb, yb, semi = bufs[p]
        dx = pltpu.async_copy(x_hbm.at[pl.ds(base + ch * CH, CH)], xb, semi)
        dy = pltpu.async_copy(y_hbm.at[pl.ds(base + ch * CH, CH)], yb, semi)
        descs_in[p] = (dx, dy)

    start_in(0)
    _zero_ref(h1, NB1 * L)
    for ch in range(NCH):
        p = ch & 1
        xb, yb, _semi = bufs[p]
        dx, dy = descs_in[p]
        dx.wait()
        dy.wait()
        if ch + 1 < NCH:
            start_in(ch + 1)
        cbase = ch * CH

        def body(i, _):
            off = i * (U * L)
            xs = [xb[pl.ds(off + u * L, L)] for u in range(U)]
            ys = [yb[pl.ds(off + u * L, L)] for u in range(U)]
            es = [jnp.exp(-jnp.abs(x)) for x in xs]
            l1 = [e * (0.99988787
                       + e * (-0.49636774
                              + e * (0.30467086
                                     + e * (-0.15602694 + e * 0.04106407))))
                  for e in es]
            bces = [jnp.maximum(x, 0.0) - x * y + l
                    for x, y, l in zip(xs, ys, l1)]
            for u in range(U):
                bce[pl.ds(cbase + off + u * L, L)] = bces[u]
            sh21 = jnp.full((L,), 21, jnp.int32)
            for u in range(U):
                key = plsc.bitcast(bces[u], jnp.int32)
                b1 = lax.shift_right_logical(key, sh21)
                idx = b1 + lane * NB1
                plsc.addupdate_scatter(h1, [idx], ones)
            return 0

        lax.fori_loop(0, CH // (U * L), body, 0)
    _tree_merge_rows(h1, L, NB1, m1c, NB1)
    pltpu.sync_copy(m1c, shc.at[pl.ds(sid * NB1, NB1)])
    plsc.subcore_barrier()
    pltpu.sync_copy(shc, h1)
    _tree_merge_rows(h1, NS, NB1, m1c, NB1)
    b1_splat = _find_bin(m1c, jnp.float32(float(K)), NB1)
    _zero_ref(h2c, NB2)
    _zero_ref(h2s, NB2)
    sh21 = jnp.full((L,), 21, jnp.int32)
    sh9 = jnp.full((L,), 9, jnp.int32)
    m12 = jnp.full((L,), 0xFFF, jnp.int32)
    zero16 = jnp.zeros((L,), jnp.float32)

    def body2(i, acc):
        off = i * (U * L)
        bces = [bce[pl.ds(off + u * L, L)] for u in range(U)]
        for u in range(U):
            key = plsc.bitcast(bces[u], jnp.int32)
            b1 = lax.shift_right_logical(key, sh21)
            acc = acc + jnp.where(b1 > b1_splat, bces[u], zero16)
            mask = b1 == b1_splat
            b2 = jnp.bitwise_and(lax.shift_right_logical(key, sh9), m12)
            plsc.addupdate_scatter(h2c, [b2], ones, mask=mask)
            plsc.addupdate_scatter(h2s, [b2], bces[u], mask=mask)
        return acc

    acc_sab = lax.fori_loop(0, M // (U * L), body2, zero16)
    obuf[...] = acc_sab
    pltpu.sync_copy(obuf, shs.at[pl.ds(sid * L, L)])
    pltpu.sync_copy(h2c, sh2c.at[pl.ds(sid * NB2, NB2)])
    pltpu.sync_copy(h2s, sh2s.at[pl.ds(sid * NB2, NB2)])
    plsc.subcore_barrier()

    @pl.when((sid == 0) & (cid == 0))
    def _():
        pltpu.sync_copy(shs.at[pl.ds(0, NS * L)], h1.at[pl.ds(0, NS * L)])
        _tree_merge_rows(h1, NS, L, m1s, L)
        s_ab = jnp.sum(m1s[pl.ds(0, L)])
        pltpu.sync_copy(sh2c, bce.at[pl.ds(0, NS * NB2)])
        _tree_merge_rows(bce, NS, NB2, g2c, NB2)
        pltpu.sync_copy(sh2s, bce.at[pl.ds(0, NS * NB2)])
        _tree_merge_rows(bce, NS, NB2, g2s, NB2)
        c_ab, _sg, _c1, _s1 = _select_bin(m1c, m1c, jnp.float32(float(K)),
                                          NB1)
        t2 = jnp.float32(float(K)) - c_ab
        c_hi2, s_hi2, c_str, s_str = _select_bin(g2c, g2s, t2, NB2)
        kf = jnp.full((L,), float(K), jnp.float32)
        c_hi = ones * c_ab + ones * c_hi2
        s_hi = ones * s_ab + ones * s_hi2
        borrow = (kf - c_hi) * (ones * s_str) / jnp.maximum(ones * c_str, ones)
        loss = (s_hi + borrow) / kf
        obuf[...] = loss
        pltpu.sync_copy(obuf, loss_hbm)


_kern = functools.partial(
    pl.kernel,
    out_type=jax.ShapeDtypeStruct((L,), jnp.float32),
    mesh=_mesh,
    compiler_params=_cparams,
    scratch_types=[pltpu.VMEM((CH,), jnp.float32),
                   pltpu.VMEM((CH,), jnp.float32),
                   pltpu.VMEM((CH,), jnp.float32),
                   pltpu.VMEM((CH,), jnp.float32),
                   pltpu.VMEM((M,), jnp.float32),
                   pltpu.VMEM((NB1 * L,), jnp.float32),
                   pltpu.VMEM((NB1,), jnp.float32),
                   pltpu.VMEM((NB1,), jnp.float32),
                   pltpu.VMEM((NB2,), jnp.float32),
                   pltpu.VMEM((NB2,), jnp.float32),
                   pltpu.VMEM((NB2,), jnp.float32),
                   pltpu.VMEM((NB2,), jnp.float32),
                   pltpu.VMEM_SHARED((NS * NB1,), jnp.float32),
                   pltpu.VMEM_SHARED((NS * NB1,), jnp.float32),
                   pltpu.VMEM_SHARED((NS * NB2,), jnp.float32),
                   pltpu.VMEM_SHARED((NS * NB2,), jnp.float32),
                   pltpu.VMEM((L,), jnp.float32),
                   pltpu.SemaphoreType.DMA,
                   pltpu.SemaphoreType.DMA],
)(_body)


def kernel(task_score_head, task_score_labels, task_agn_idx):
    del task_agn_idx
    x = task_score_head.reshape(N)
    y = task_score_labels.reshape(N)
    loss_vec = _kern(x, y)
    return loss_vec[0]


# speedup vs baseline: 1.1647x; 1.1647x over previous
import functools

import jax
import jax.numpy as jnp
from jax import lax
from jax.experimental import pallas as pl
from jax.experimental.pallas import tpu as pltpu
from jax.experimental.pallas import tpu_sc as plsc

N = 1048576
K = 4096
NC = 2
NS = 16
NW = NC * NS
L = 16
M = N // NW
MB = N // NS
CH = 8192
NCH = M // CH
NCHB = MB // CH
NB = 2048
NBV = NB // L
U = 8

_mesh = plsc.VectorSubcoreMesh(core_axis_name="c", subcore_axis_name="s")
_cparams = pltpu.CompilerParams(needs_layout_passes=False)


def _keybins(bce):
    key = plsc.bitcast(bce, jnp.int32)
    sh20 = jnp.full((L,), 20, jnp.int32)
    sh9 = jnp.full((L,), 9, jnp.int32)
    m11 = jnp.full((L,), 0x7FF, jnp.int32)
    b1 = lax.shift_right_logical(key, sh20)
    b2 = jnp.bitwise_and(lax.shift_right_logical(key, sh9), m11)
    return b1, b2


def _zero_ref(ref, n_words):
    zero16 = jnp.zeros((L,), jnp.float32)
    blk = 32 * L

    def z(i, _):
        for u in range(32):
            ref[pl.ds(i * blk + u * L, L)] = zero16
        return 0

    lax.fori_loop(0, n_words // blk, z, 0)


def _tree_merge_rows(src, n_rows, row_stride, dst):
    def merge(j, _):
        vs = [src[pl.ds(r * row_stride + j * L, L)] for r in range(n_rows)]
        while len(vs) > 1:
            vs = [vs[i] + vs[i + 1] for i in range(0, len(vs) - 1, 2)] + (
                [vs[-1]] if len(vs) % 2 else [])
        dst[pl.ds(j * L, L)] = vs[0]
        return 0

    lax.fori_loop(0, NBV, merge, 0)


def _find_bin(mc, threshold):
    def body(jj, carry):
        cnt_acc, sum_carry = carry
        j = NBV - 1 - jj
        v = mc[pl.ds(j * L, L)]
        sfx = lax.rev(jnp.cumsum(lax.rev(v, (0,))), (0,)) + sum_carry
        ge = sfx >= threshold
        cnt_acc = cnt_acc + plsc.all_reduce_population_count(ge)
        return cnt_acc, sum_carry + jnp.sum(v)

    cnt, _ = lax.fori_loop(
        0, NBV, body, (jnp.zeros((L,), jnp.int32), jnp.float32(0.0))
    )
    return cnt - 1


def _select_bin(mc, ms, threshold):
    zero = jnp.zeros((L,), jnp.float32)

    def body(jj, carry):
        sum_carry, cgt, sgt, ceq, seq = carry
        j = NBV - 1 - jj
        vc = mc[pl.ds(j * L, L)]
        vs = ms[pl.ds(j * L, L)]
        sfx = lax.rev(jnp.cumsum(lax.rev(vc, (0,))), (0,)) + sum_carry
        ge = sfx >= threshold
        lt = sfx < threshold
        eq = jnp.logical_and(ge, (sfx - vc) < threshold)
        return (sum_carry + jnp.sum(vc),
                cgt + jnp.where(lt, vc, zero), sgt + jnp.where(lt, vs, zero),
                ceq + jnp.where(eq, vc, zero), seq + jnp.where(eq, vs, zero))

    _, cgt, sgt, ceq, seq = lax.fori_loop(
        0, NBV, body, (jnp.float32(0.0), zero, zero, zero, zero))
    return jnp.sum(cgt), jnp.sum(sgt), jnp.sum(ceq), jnp.sum(seq)


def _pass1_body(x_hbm, y_hbm, bce_hbm, h1_hbm,
                xbuf0, ybuf0, bbuf0, xbuf1, ybuf1, bbuf1,
                hc, hs, mc, ms, shc, shs, semi0, semi1, semo0, semo1):
    cid = lax.axis_index("c")
    sid = lax.axis_index("s")
    wid = sid * NC + cid
    base = wid * M
    lane = lax.iota(jnp.int32, L)
    bufs = [(xbuf0, ybuf0, bbuf0, semi0, semo0),
            (xbuf1, ybuf1, bbuf1, semi1, semo1)]
    descs_in = [None, None]
    descs_out = [None, None]

    def start_in(ch):
        p = ch & 1
        xb, yb, _bb, semi, _semo = bufs[p]
        dx = pltpu.async_copy(x_hbm.at[pl.ds(base + ch * CH, CH)], xb, semi)
        dy = pltpu.async_copy(y_hbm.at[pl.ds(base + ch * CH, CH)], yb, semi)
        descs_in[p] = (dx, dy)

    start_in(0)
    _zero_ref(hc, NB * L)
    _zero_ref(hs, NB * L)
    for ch in range(NCH):
        p = ch & 1
        xb, yb, bb, _semi, semo = bufs[p]
        dx, dy = descs_in[p]
        dx.wait()
        dy.wait()
        if ch + 1 < NCH:
            start_in(ch + 1)
        if descs_out[p] is not None:
            descs_out[p].wait()

        def body(i, _):
            off = i * (U * L)
            xs = [xb[pl.ds(off + u * L, L)] for u in range(U)]
            ys = [yb[pl.ds(off + u * L, L)] for u in range(U)]
            es = [jnp.exp(-jnp.abs(x)) for x in xs]
            l1 = [e * (0.99988787
                       + e * (-0.49636774
                              + e * (0.30467086
                                     + e * (-0.15602694 + e * 0.04106407))))
                  for e in es]
            bces = [jnp.maximum(x, 0.0) - x * y + l
                    for x, y, l in zip(xs, ys, l1)]
            for u in range(U):
                bb[pl.ds(off + u * L, L)] = bces[u]
            ones = jnp.ones((L,), jnp.float32)
            for u in range(U):
                b1, _b2 = _keybins(bces[u])
                idx = b1 + lane * NB
                plsc.addupdate_scatter(hc, [idx], ones)
                plsc.addupdate_scatter(hs, [idx], bces[u])
            return 0

        lax.fori_loop(0, CH // (U * L), body, 0)
        descs_out[p] = pltpu.async_copy(
            bb, bce_hbm.at[pl.ds(base + ch * CH, CH)], semo)
    for p in range(2):
        if descs_out[p] is not None:
            descs_out[p].wait()
    _tree_merge_rows(hc, L, NB, mc)
    _tree_merge_rows(hs, L, NB, ms)
    pltpu.sync_copy(mc, shc.at[pl.ds(sid * NB, NB)])
    pltpu.sync_copy(ms, shs.at[pl.ds(sid * NB, NB)])
    plsc.subcore_barrier()

    @pl.when(sid == 0)
    def _():
        pltpu.sync_copy(shc, hc)
        _tree_merge_rows(hc, NS, NB, mc)
        pltpu.sync_copy(mc, h1_hbm.at[pl.ds(cid * NB, NB)])

    @pl.when(sid == 1)
    def _():
        pltpu.sync_copy(shs, hc)
        _tree_merge_rows(hc, NS, NB, mc)
        pltpu.sync_copy(mc, h1_hbm.at[pl.ds((2 + cid) * NB, NB)])


def _passB_body(bce_hbm, h1_hbm, loss_hbm,
                bbuf0, bbuf1, tbuf, hc, hs, mc, ms, g2c, g2s, stag,
                shc, shs, obuf, semi0, semi1):
    cid = lax.axis_index("c")
    sid = lax.axis_index("s")
    base = sid * MB
    bufs = [(bbuf0, semi0), (bbuf1, semi1)]
    descs_in = [None, None]

    def start_in(ch):
        p = ch & 1
        bb, semi = bufs[p]
        descs_in[p] = pltpu.async_copy(
            bce_hbm.at[pl.ds(base + ch * CH, CH)], bb, semi)

    start_in(0)
    pltpu.sync_copy(h1_hbm, tbuf)

    def acc(j, _):
        mc[pl.ds(j * L, L)] = (tbuf[pl.ds(j * L, L)]
                               + tbuf[pl.ds(NB + j * L, L)])
        ms[pl.ds(j * L, L)] = (tbuf[pl.ds(2 * NB + j * L, L)]
                               + tbuf[pl.ds(3 * NB + j * L, L)])
        return 0

    lax.fori_loop(0, NBV, acc, 0, unroll=4)
    b1_splat = _find_bin(mc, jnp.float32(float(K)))
    _zero_ref(hc, NB)
    _zero_ref(hs, NB)
    for ch in range(NCHB):
        p = ch & 1
        bb, _semi = bufs[p]
        descs_in[p].wait()
        if ch + 1 < NCHB:
            start_in(ch + 1)

        def body(i, _):
            off = i * (U * L)
            bces = [bb[pl.ds(off + u * L, L)] for u in range(U)]
            ones = jnp.ones((L,), jnp.float32)
            for u in range(U):
                b1, b2 = _keybins(bces[u])
                mask = b1 == b1_splat
                plsc.addupdate_scatter(hc, [b2], ones, mask=mask)
                plsc.addupdate_scatter(hs, [b2], bces[u], mask=mask)
            return 0

        lax.fori_loop(0, CH // (U * L), body, 0)
    pltpu.sync_copy(hc, shc.at[pl.ds(sid * NB, NB)])
    pltpu.sync_copy(hs, shs.at[pl.ds(sid * NB, NB)])
    plsc.subcore_barrier()

    @pl.when((sid == 1) & (cid == 0))
    def _():
        pltpu.sync_copy(shs, stag)
        _tree_merge_rows(stag, NS, NB, g2s)
        pltpu.sync_copy(g2s, shs.at[pl.ds(0, NB)])

    @pl.when((sid == 0) & (cid == 0))
    def _():
        pltpu.sync_copy(shc, stag)
        _tree_merge_rows(stag, NS, NB, g2c)

    plsc.subcore_barrier()

    @pl.when((sid == 0) & (cid == 0))
    def _():
        pltpu.sync_copy(shs.at[pl.ds(0, NB)], g2s)
        c_ab, s_ab, _c1, _s1 = _select_bin(mc, ms, jnp.float32(float(K)))
        t2 = jnp.float32(float(K)) - c_ab
        c_hi2, s_hi2, c_str, s_str = _select_bin(g2c, g2s, t2)
        ones = jnp.ones((L,), jnp.float32)
        kf = jnp.full((L,), float(K), jnp.float32)
        c_hi = ones * c_ab + ones * c_hi2
        s_hi = ones * s_ab + ones * s_hi2
        borrow = (kf - c_hi) * (ones * s_str) / jnp.maximum(ones * c_str, ones)
        loss = (s_hi + borrow) / kf
        obuf[...] = loss
        pltpu.sync_copy(obuf, loss_hbm)


_pass1 = functools.partial(
    pl.kernel,
    out_type=[jax.ShapeDtypeStruct((N,), jnp.float32),
              jax.ShapeDtypeStruct((4 * NB,), jnp.float32)],
    mesh=_mesh,
    compiler_params=_cparams,
    scratch_types=[pltpu.VMEM((CH,), jnp.float32),
                   pltpu.VMEM((CH,), jnp.float32),
                   pltpu.VMEM((CH,), jnp.float32),
                   pltpu.VMEM((CH,), jnp.float32),
                   pltpu.VMEM((CH,), jnp.float32),
                   pltpu.VMEM((CH,), jnp.float32),
                   pltpu.VMEM((NB * L,), jnp.float32),
                   pltpu.VMEM((NB * L,), jnp.float32),
                   pltpu.VMEM((NB,), jnp.float32),
                   pltpu.VMEM((NB,), jnp.float32),
                   pltpu.VMEM_SHARED((NS * NB,), jnp.float32),
                   pltpu.VMEM_SHARED((NS * NB,), jnp.float32),
                   pltpu.SemaphoreType.DMA,
                   pltpu.SemaphoreType.DMA,
                   pltpu.SemaphoreType.DMA,
                   pltpu.SemaphoreType.DMA],
)(_pass1_body)

_passB = functools.partial(
    pl.kernel,
    out_type=jax.ShapeDtypeStruct((L,), jnp.float32),
    mesh=_mesh,
    compiler_params=_cparams,
    scratch_types=[pltpu.VMEM((CH,), jnp.float32),
                   pltpu.VMEM((CH,), jnp.float32),
                   pltpu.VMEM((4 * NB,), jnp.float32),
                   pltpu.VMEM((NB,), jnp.float32),
                   pltpu.VMEM((NB,), jnp.float32),
                   pltpu.VMEM((NB,), jnp.float32),
                   pltpu.VMEM((NB,), jnp.float32),
                   pltpu.VMEM((NB,), jnp.float32),
                   pltpu.VMEM((NB,), jnp.float32),
                   pltpu.VMEM((NS * NB,), jnp.float32),
                   pltpu.VMEM_SHARED((NS * NB,), jnp.float32),
                   pltpu.VMEM_SHARED((NS * NB,), jnp.float32),
                   pltpu.VMEM((L,), jnp.float32),
                   pltpu.SemaphoreType.DMA,
                   pltpu.SemaphoreType.DMA],
)(_passB_body)


def kernel(task_score_head, task_score_labels, task_agn_idx):
    del task_agn_idx
    x = task_score_head.reshape(N)
    y = task_score_labels.reshape(N)
    bce, h1 = _pass1(x, y)
    loss_vec = _passB(bce, h1)
    return loss_vec[0]
